# baseline (speedup 1.0000x reference)
import functools

import jax
import jax.numpy as jnp
from jax import lax
from jax.experimental import pallas as pl
from jax.experimental.pallas import tpu as pltpu
from jax.experimental.pallas import tpu_sc as plsc

EPS = 1e-12
L = 16
SPG = 2


def _rsqrt16(v):
    bits = lax.bitcast_convert_type(v, jnp.int32)
    y = lax.bitcast_convert_type(
        jnp.int32(0x5F3759DF) - lax.shift_right_logical(bits, 1), jnp.float32)
    for _ in range(3):
        y = y * (1.5 - 0.5 * v * y * y)
    return y


def _make_sc_kernel(B, S, H, V):
    info = plsc.get_sparse_core_info()
    NC, NS = info.num_cores, info.num_subcores
    NW = NC * NS
    BBLK = B // NW
    HB = H // 8
    NG = BBLK // L
    G = S // SPG
    assert B % NW == 0 and BBLK == 128 and H % L == 0
    assert S % SPG == 0 and G % 2 == 0

    mesh = plsc.VectorSubcoreMesh(core_axis_name="c", subcore_axis_name="s")

    @functools.partial(
        pl.kernel,
        mesh=mesh,
        compiler_params=pltpu.CompilerParams(use_tc_tiling_on_sc=False,
                                             needs_layout_passes=False),
        out_type=jax.ShapeDtypeStruct((S, HB, NW, 8, BBLK), jnp.float32),
        scratch_types=[
            pltpu.VMEM((SPG, BBLK), jnp.int32),
            pltpu.VMEM((SPG, BBLK), jnp.int32),
            pltpu.VMEM((SPG, BBLK, H), jnp.float32),
            pltpu.VMEM((SPG, BBLK, H), jnp.float32),
            pltpu.VMEM((SPG, HB, 8, BBLK), jnp.float32),
            pltpu.VMEM((SPG, HB, 8, BBLK), jnp.float32),
            pltpu.VMEM((SPG * H * L,), jnp.float32),
            pltpu.VMEM((SPG * H * L,), jnp.float32),
            pltpu.VMEM((H * L,), jnp.float32),
            pltpu.VMEM((H * L,), jnp.float32),
            pltpu.VMEM((L,), jnp.float32),
            pltpu.SemaphoreType.DMA,
            pltpu.SemaphoreType.DMA,
            pltpu.SemaphoreType.DMA,
            pltpu.SemaphoreType.DMA,
            pltpu.SemaphoreType.DMA,
            pltpu.SemaphoreType.DMA,
            pltpu.SemaphoreType.DMA,
            pltpu.SemaphoreType.DMA,
        ],
    )
    def k(idsT, wemb, abct_h, w0_h, gbc_h, bbc_h, out_h,
          ib0, ib1, rb0, rb1, ob0, ob1, ab0, ab1, gbc, bbc, w0_v,
          is0, is1, gs0, gs1, ws0, ws1, as0, as1):
        wid = lax.axis_index("s") * NC + lax.axis_index("c")
        b0 = wid * BBLK
        pltpu.sync_copy(w0_h, w0_v)
        pltpu.sync_copy(gbc_h, gbc)
        pltpu.sync_copy(bbc_h, bbc)

        lanes = lax.iota(jnp.int32, L)
        zero16 = lanes ^ lanes
        w0 = w0_v[...]
        zf = zero16.astype(jnp.float32)
        rowidx = [lanes + lg * L for lg in range(NG)]
        inv_h = 1.0 / H

        ibs = (ib0, ib1)
        rbs = (rb0, rb1)
        obs = (ob0, ob1)
        abs_ = (ab0, ab1)
        isems = (is0, is1)
        gsems = (gs0, gs1)
        wsems = (ws0, ws1)
        asems = (as0, as1)

        def idx_desc(g, par):
            return pltpu.make_async_copy(
                idsT.at[pl.ds(g * SPG, SPG), pl.ds(b0, BBLK)],
                ibs[par], isems[par])

        def ab_desc(g, par):
            return pltpu.make_async_copy(
                abct_h.at[pl.ds(g * SPG * H * L, SPG * H * L)],
                abs_[par], asems[par])

        def gather_descs(par):
            descs = []
            for sg in range(SPG):
                for lg in range(NG):
                    iv = ibs[par][sg, pl.ds(lg * L, L)]
                    descs.append(pltpu.make_async_copy(
                        wemb.at[iv], rbs[par].at[sg, pl.ds(lg * L, L)],
                        gsems[par]))
            return descs

        def write_desc(g, par):
            return pltpu.make_async_copy(
                obs[par], out_h.at[pl.ds(g * SPG, SPG), :, wid], wsems[par])

        pltpu.sync_copy(idsT.at[pl.ds(0, SPG), pl.ds(b0, BBLK)], ib0)
        for d in gather_descs(0):
            d.start()
        ab_desc(0, 0).start()
        idx_desc(1, 1).start()
        ab_desc(1, 1).start()

        def make_sg_body(rows_v, ab_v, ob_v):
            def sg_body(sg, _):
                sgsplat = zero16 + sg
                abase = sg * H * L
                acc = [zf] * NG
                acc2 = [zf] * NG
                for h in range(H):
                    a_h = ab_v[pl.ds(abase + h * L, L)]
                    hvec = zero16 + h
                    hb, hi = h // 8, h % 8
                    for lg in range(NG):
                        v = plsc.load_gather(
                            rows_v, [sgsplat, rowidx[lg], hvec])
                        x = v * w0 + a_h
                        ob_v[sg, hb, hi, pl.ds(lg * L, L)] = x
                        acc[lg] = acc[lg] + x
                        acc2[lg] = x * x + acc2[lg]
                means, scales = [], []
                for lg in range(NG):
                    mean = acc[lg] * inv_h
                    var = acc2[lg] * inv_h - mean * mean
                    means.append(mean)
                    scales.append(_rsqrt16(var + EPS))
                def norm_block(hb, _n):
                    for hi in range(8):
                        off = (hb * 8 + hi) * L
                        gh = gbc[pl.ds(off, L)]
                        bh = bbc[pl.ds(off, L)]
                        for lg in range(NG):
                            x = ob_v[sg, hb, hi, pl.ds(lg * L, L)]
                            o = (x - means[lg]) * (scales[lg] * gh) + bh
                            ob_v[sg, hb, hi, pl.ds(lg * L, L)] = o
                    return _n

                lax.fori_loop(0, HB, norm_block, None)
                return _
            return sg_body

        def step(gg, g, par):
            nxt = 1 - par

            def fire_next_gather():
                idx_desc(g + 1, nxt).wait()
                for d in gather_descs(nxt):
                    d.start()

            if par == 0:
                fire_next_gather()
            else:
                pl.when(gg < G // 2 - 1)(fire_next_gather)

            for d in gather_descs(par):
                d.wait()
            ab_desc(g, par).wait()

            @pl.when(gg < G // 2 - 1)
            def _():
                idx_desc(g + 2, par).start()

            @pl.when(g > 1)
            def _():
                write_desc(g, par).wait()

            lax.fori_loop(0, SPG,
                          make_sg_body(rbs[par], abs_[par], obs[par]), None)

            write_desc(g, par).start()

            @pl.when(gg < G // 2 - 1)
            def _():
                ab_desc(g + 2, par).start()

        def pair(gg, _):
            step(gg, 2 * gg, 0)
            step(gg, 2 * gg + 1, 1)
            return _

        lax.fori_loop(0, G // 2, pair, None)

        write_desc(0, 0).wait()
        write_desc(1, 1).wait()

    return k


def kernel(input_ids, word_emb, pos_emb, type_emb, conv_w, ln_gamma, ln_beta):
    B, S = input_ids.shape
    V, H = word_emb.shape
    w = conv_w.reshape(2).astype(jnp.float32)
    atab = w[1] * pos_emb[:S] + type_emb[0]
    abct = jnp.broadcast_to(atab[:, :, None], (S, H, L)).reshape(S * H * L)
    gbct = jnp.broadcast_to(
        ln_gamma.astype(jnp.float32)[:, None], (H, L)).reshape(H * L)
    bbct = jnp.broadcast_to(
        ln_beta.astype(jnp.float32)[:, None], (H, L)).reshape(H * L)
    w0v = jnp.full((L,), w[0], jnp.float32)
    idsT = input_ids.T.astype(jnp.int32)
    out5d = _make_sc_kernel(B, S, H, V)(idsT, word_emb, abct, w0v, gbct, bbct)
    return jnp.transpose(out5d, (2, 4, 0, 1, 3)).reshape(B, S, H)


# speedup vs baseline: 2.1008x; 2.1008x over previous
import functools

import jax
import jax.numpy as jnp
from jax import lax
from jax.experimental import pallas as pl
from jax.experimental.pallas import tpu as pltpu
from jax.experimental.pallas import tpu_sc as plsc

EPS = 1e-12
L = 16
SPG = 4
UNROLL = 2


def _rsqrt16(v):
    bits = lax.bitcast_convert_type(v, jnp.int32)
    y = lax.bitcast_convert_type(
        jnp.int32(0x5F3759DF) - lax.shift_right_logical(bits, 1), jnp.float32)
    for _ in range(3):
        y = y * (1.5 - 0.5 * v * y * y)
    return y


def _make_sc_kernel(B, S, H, V):
    info = plsc.get_sparse_core_info()
    NC, NS = info.num_cores, info.num_subcores
    NW = NC * NS
    BBLK = B // NW
    HB = H // 8
    NG = BBLK // L
    G = S // SPG
    assert B % NW == 0 and BBLK == 128 and H % L == 0
    assert S % SPG == 0 and G % 2 == 0

    mesh = plsc.VectorSubcoreMesh(core_axis_name="c", subcore_axis_name="s")

    @functools.partial(
        pl.kernel,
        mesh=mesh,
        compiler_params=pltpu.CompilerParams(use_tc_tiling_on_sc=False,
                                             needs_layout_passes=False),
        out_type=jax.ShapeDtypeStruct((S, HB, NW, 8, BBLK), jnp.float32),
        scratch_types=[
            pltpu.VMEM((SPG, BBLK), jnp.int32),
            pltpu.VMEM((SPG, BBLK), jnp.int32),
            pltpu.VMEM((SPG, BBLK, H), jnp.float32),
            pltpu.VMEM((SPG, BBLK, H), jnp.float32),
            pltpu.VMEM((HB, 8, BBLK), jnp.float32),
            pltpu.VMEM((HB, 8, BBLK), jnp.float32),
            pltpu.VMEM((SPG * H * L,), jnp.float32),
            pltpu.VMEM((SPG * H * L,), jnp.float32),
            pltpu.VMEM((H * L,), jnp.float32),
            pltpu.VMEM((H * L,), jnp.float32),
            pltpu.VMEM((L,), jnp.float32),
            pltpu.SemaphoreType.DMA,
            pltpu.SemaphoreType.DMA,
            pltpu.SemaphoreType.DMA,
            pltpu.SemaphoreType.DMA,
            pltpu.SemaphoreType.DMA,
            pltpu.SemaphoreType.DMA,
            pltpu.SemaphoreType.DMA,
            pltpu.SemaphoreType.DMA,
        ],
    )
    def k(idsT, wemb, abct_h, w0_h, gbc_h, bbc_h, out_h,
          ib0, ib1, rb0, rb1, ob0, ob1, ab0, ab1, gbc, bbc, w0_v,
          is0, is1, gs0, gs1, ws0, ws1, as0, as1):
        wid = lax.axis_index("s") * NC + lax.axis_index("c")
        b0 = wid * BBLK
        pltpu.sync_copy(w0_h, w0_v)
        pltpu.sync_copy(gbc_h, gbc)
        pltpu.sync_copy(bbc_h, bbc)

        lanes = lax.iota(jnp.int32, L)
        zero16 = lanes ^ lanes
        w0 = w0_v[...]
        zf = zero16.astype(jnp.float32)
        rowidx = [lanes + lg * L for lg in range(NG)]
        inv_h = 1.0 / H

        ibs = (ib0, ib1)
        rbs = (rb0, rb1)
        obs = (ob0, ob1)
        abs_ = (ab0, ab1)
        isems = (is0, is1)
        gsems = (gs0, gs1)
        wsems = (ws0, ws1)
        asems = (as0, as1)

        def idx_desc(g, par):
            return pltpu.make_async_copy(
                idsT.at[pl.ds(g * SPG, SPG), pl.ds(b0, BBLK)],
                ibs[par], isems[par])

        def ab_desc(g, par):
            return pltpu.make_async_copy(
                abct_h.at[pl.ds(g * SPG * H * L, SPG * H * L)],
                abs_[par], asems[par])

        def gather_descs(par):
            descs = []
            for sg in range(SPG):
                for lg in range(NG):
                    iv = ibs[par][sg, pl.ds(lg * L, L)]
                    descs.append(pltpu.make_async_copy(
                        wemb.at[iv], rbs[par].at[sg, pl.ds(lg * L, L)],
                        gsems[par]))
            return descs

        def write_desc(s, par):
            return pltpu.make_async_copy(
                obs[par], out_h.at[s, :, wid], wsems[par])

        pltpu.sync_copy(idsT.at[pl.ds(0, SPG), pl.ds(b0, BBLK)], ib0)
        for d in gather_descs(0):
            d.start()
        ab_desc(0, 0).start()
        idx_desc(1, 1).start()
        ab_desc(1, 1).start()

        def compute_s(s, sg, rows_v, ab_v, ob_v):
            sgsplat = zero16 + sg
            abase = sg * H * L

            def ph1(h, carry):
                accs = list(carry)
                a_h = ab_v[pl.ds(abase + h * L, L)]
                hsplat = jnp.full((L,), h, jnp.int32)
                hb = h // 8
                hi = h % 8
                for lg in range(NG):
                    v = plsc.load_gather(rows_v, [sgsplat, rowidx[lg], hsplat])
                    x = v * w0 + a_h
                    ob_v[hb, hi, pl.ds(lg * L, L)] = x
                    accs[2 * lg] = accs[2 * lg] + x
                    accs[2 * lg + 1] = x * x + accs[2 * lg + 1]
                return tuple(accs)

            stats = plsc.parallel_loop(0, H, unroll=UNROLL,
                                       carry=tuple([zf] * (2 * NG)))(ph1)

            means, scales = [], []
            for lg in range(NG):
                mean = stats[2 * lg] * inv_h
                var = stats[2 * lg + 1] * inv_h - mean * mean
                means.append(mean)
                scales.append(_rsqrt16(var + EPS))

            def ph3(h):
                gh = gbc[pl.ds(h * L, L)]
                bh = bbc[pl.ds(h * L, L)]
                hb = h // 8
                hi = h % 8
                for lg in range(NG):
                    x = ob_v[hb, hi, pl.ds(lg * L, L)]
                    o = (x - means[lg]) * (scales[lg] * gh) + bh
                    ob_v[hb, hi, pl.ds(lg * L, L)] = o

            plsc.parallel_loop(0, H, unroll=UNROLL)(ph3)

        def step(gg, g, par):
            nxt = 1 - par

            def fire_next_gather():
                idx_desc(g + 1, nxt).wait()
                for d in gather_descs(nxt):
                    d.start()

            if par == 0:
                fire_next_gather()
            else:
                pl.when(gg < G // 2 - 1)(fire_next_gather)

            for d in gather_descs(par):
                d.wait()
            ab_desc(g, par).wait()

            @pl.when(gg < G // 2 - 1)
            def _():
                idx_desc(g + 2, par).start()

            rows_v = rbs[par]
            ab_v = abs_[par]
            for sg in range(SPG):
                s = g * SPG + sg
                opar = sg % 2
                ob_v = obs[opar]

                @pl.when(s > 1)
                def _():
                    write_desc(s, opar).wait()

                compute_s(s, sg, rows_v, ab_v, ob_v)
                write_desc(s, opar).start()

            @pl.when(gg < G // 2 - 1)
            def _():
                ab_desc(g + 2, par).start()

        def pair(gg, _):
            step(gg, 2 * gg, 0)
            step(gg, 2 * gg + 1, 1)
            return _

        lax.fori_loop(0, G // 2, pair, None)

        write_desc(0, 0).wait()
        write_desc(1, 1).wait()

    return k


def kernel(input_ids, word_emb, pos_emb, type_emb, conv_w, ln_gamma, ln_beta):
    B, S = input_ids.shape
    V, H = word_emb.shape
    w = conv_w.reshape(2).astype(jnp.float32)
    atab = w[1] * pos_emb[:S] + type_emb[0]
    abct = jnp.broadcast_to(atab[:, :, None], (S, H, L)).reshape(S * H * L)
    gbct = jnp.broadcast_to(
        ln_gamma.astype(jnp.float32)[:, None], (H, L)).reshape(H * L)
    bbct = jnp.broadcast_to(
        ln_beta.astype(jnp.float32)[:, None], (H, L)).reshape(H * L)
    w0v = jnp.full((L,), w[0], jnp.float32)
    idsT = input_ids.T.astype(jnp.int32)
    out5d = _make_sc_kernel(B, S, H, V)(idsT, word_emb, abct, w0v, gbct, bbct)
    return jnp.transpose(out5d, (2, 4, 0, 1, 3)).reshape(B, S, H)
